# baseline (speedup 1.0000x reference)
import functools

import jax
import jax.numpy as jnp
from jax import lax
from jax.experimental import pallas as pl
from jax.experimental.pallas import tpu as pltpu
from jax.experimental.pallas import tpu_sc as plsc

_V = 1_000_000
_B = 4096 * 200
_D = 64
_DP = 128
_NC = 2
_NS = 16
_NW = _NC * _NS
_L = 16


_RB = 128
_JPW = 246


def _transpose_body(wt_hbm, tail_hbm, tp_hbm, sbuf0, sbuf1, obuf0, obuf1,
                    tbuf, isem0, isem1, osem0, osem1):
    wid = lax.axis_index("s") * _NC + lax.axis_index("c")
    sbufs, obufs = (sbuf0, sbuf1), (obuf0, obuf1)
    isems, osems = (isem0, isem1), (osem0, osem1)

    _LAST = (_V // _RB - 1) * _RB

    def r0_of(j):
        return jnp.minimum((wid + _NW * j) * _RB, _LAST)

    def in_copy(j, s):
        return pltpu.make_async_copy(
            wt_hbm.at[:, pl.ds(r0_of(j), _RB)], sbufs[s], isems[s])

    def out_copy(j, s):
        return pltpu.make_async_copy(
            obufs[s], tp_hbm.at[pl.ds(r0_of(j), _RB)], osems[s])

    iota = lax.iota(jnp.int32, _L)
    row_sets = [iota + _L * k for k in range(_D // _L)]

    _U = 8

    def transpose_block(s):
        sbuf, obuf = sbufs[s], obufs[s]

        def tbody(g, carry):
            base = g * _U
            vals = []
            for u in range(_U):
                cols = jnp.full((_L,), base + u, jnp.int32)
                for k in range(_D // _L):
                    vals.append(
                        (u, k, plsc.load_gather(sbuf, [row_sets[k], cols])))
            for u, k, v in vals:
                obuf[base + u, pl.ds(_L * k, _L)] = v
            return carry

        lax.fori_loop(0, _RB // _U, tbody, 0)

    in_copy(0, 0).start()

    def step(jj, carry):
        for p in range(2):
            j = jj * 2 + p
            s = p

            @pl.when(j + 1 < _JPW)
            def _():
                in_copy(j + 1, 1 - s).start()

            in_copy(j, s).wait()

            @pl.when(j >= 2)
            def _():
                out_copy(j - 2, s).wait()

            transpose_block(s)
            out_copy(j, s).start()
        return carry

    lax.fori_loop(0, _JPW // 2, step, 0)

    out_copy(_JPW - 2, 0).wait()
    out_copy(_JPW - 1, 1).wait()

    @pl.when(wid == 0)
    def _():
        pltpu.sync_copy(tail_hbm, tbuf)

        def tailrow(j, carry):
            for k in range(_D // _L):
                obuf0[j, pl.ds(_L * k, _L)] = tbuf[j, pl.ds(_L * k, _L)]
            return carry

        lax.fori_loop(0, _D, tailrow, 0)
        pltpu.sync_copy(obuf0.at[pl.ds(0, _D)], tp_hbm.at[pl.ds(_V - _D, _D)])


@functools.partial(jax.jit, donate_argnums=())
def _relayout(wt, tail):
    mesh = plsc.VectorSubcoreMesh(core_axis_name="c", subcore_axis_name="s")
    run = functools.partial(
        pl.kernel,
        mesh=mesh,
        out_type=jax.ShapeDtypeStruct((_V, _DP), jnp.float32),
        compiler_params=pltpu.CompilerParams(needs_layout_passes=False),
        scratch_types=[
            pltpu.VMEM((_D, _RB), jnp.float32),
            pltpu.VMEM((_D, _RB), jnp.float32),
            pltpu.VMEM((_RB, _DP), jnp.float32),
            pltpu.VMEM((_RB, _DP), jnp.float32),
            pltpu.VMEM((_D, _D), jnp.float32),
            pltpu.SemaphoreType.DMA,
            pltpu.SemaphoreType.DMA,
            pltpu.SemaphoreType.DMA,
            pltpu.SemaphoreType.DMA,
        ],
    )(_transpose_body)
    return run(wt, tail)


_BPW = _B // _NW
_CHUNK = 256
_NCHUNK = _BPW // _CHUNK


def _gather_body(idx_hbm, table_hbm, out_hbm, idx_v, rows0, rows1,
                 gsem0, gsem1, wsem0, wsem1):
    wid = lax.axis_index("s") * _NC + lax.axis_index("c")
    base = wid * _BPW
    pltpu.sync_copy(idx_hbm.at[pl.ds(base, _BPW)], idx_v)

    bufs = (rows0, rows1)
    gsems = (gsem0, gsem1)
    wsems = (wsem0, wsem1)

    def gather_copy(c, b):
        return pltpu.make_async_copy(
            table_hbm.at[idx_v.at[pl.ds(c * _CHUNK, _CHUNK)]],
            bufs[b], gsems[b],
        )

    def write_copy(c, b):
        return pltpu.make_async_copy(
            bufs[b], out_hbm.at[pl.ds(base + c * _CHUNK, _CHUNK)], wsems[b],
        )

    gather_copy(0, 0).start()
    gather_copy(1, 1).start()
    gather_copy(0, 0).wait()
    write_copy(0, 0).start()

    def step(g, carry):
        for p in range(2):
            c = 1 + g * 2 + p
            b, ob = (1 + p) % 2, p % 2
            write_copy(c - 1, ob).wait()
            gather_copy(c + 1, ob).start()
            gather_copy(c, b).wait()
            write_copy(c, b).start()
        return carry

    lax.fori_loop(0, (_NCHUNK - 2) // 2, step, 0)

    c = _NCHUNK - 1
    gather_copy(c, c % 2).wait()
    write_copy(c, c % 2).start()
    write_copy(c - 1, (c - 1) % 2).wait()
    write_copy(c, c % 2).wait()


@functools.partial(jax.jit, donate_argnums=())
def _embedding_gather(x_flat, emb_weight):
    mesh = plsc.VectorSubcoreMesh(core_axis_name="c", subcore_axis_name="s")
    table_pad = _relayout(emb_weight.T, emb_weight[_V - _D:, :])
    run = functools.partial(
        pl.kernel,
        mesh=mesh,
        out_type=jax.ShapeDtypeStruct((_B, _DP), jnp.float32),
        scratch_types=[
            pltpu.VMEM((_BPW,), jnp.int32),
            pltpu.VMEM((_CHUNK, _DP), jnp.float32),
            pltpu.VMEM((_CHUNK, _DP), jnp.float32),
            pltpu.SemaphoreType.DMA,
            pltpu.SemaphoreType.DMA,
            pltpu.SemaphoreType.DMA,
            pltpu.SemaphoreType.DMA,
        ],
    )(_gather_body)
    return run(x_flat, table_pad)


def kernel(x, emb_weight):
    out = _embedding_gather(x.reshape(-1).astype(jnp.int32), emb_weight)
    return out[:, :_D].reshape(x.shape + (_D,))


# speedup vs baseline: 1.5134x; 1.5134x over previous
import functools

import jax
import jax.numpy as jnp
from jax import lax
from jax.experimental import pallas as pl
from jax.experimental.pallas import tpu as pltpu
from jax.experimental.pallas import tpu_sc as plsc

_V = 1_000_000
_B = 4096 * 200
_D = 64
_DP = 128
_NC = 2
_NS = 16
_NW = _NC * _NS
_L = 16


_RB = 128
_JPW = 246


def _transpose_body(wt_hbm, tail_hbm, tp_hbm, sbuf0, sbuf1, obuf0, obuf1,
                    tbuf, isem0, isem1, osem0, osem1):
    wid = lax.axis_index("s") * _NC + lax.axis_index("c")
    sbufs, obufs = (sbuf0, sbuf1), (obuf0, obuf1)
    isems, osems = (isem0, isem1), (osem0, osem1)

    _LAST = (_V // _RB - 1) * _RB

    def r0_of(j):
        return jnp.minimum((wid + _NW * j) * _RB, _LAST)

    def in_copy(j, s):
        return pltpu.make_async_copy(
            wt_hbm.at[:, pl.ds(r0_of(j), _RB)], sbufs[s], isems[s])

    def out_copy(j, s):
        return pltpu.make_async_copy(
            obufs[s], tp_hbm.at[pl.ds(r0_of(j), _RB)], osems[s])

    iota = lax.iota(jnp.int32, _L)
    perms = [(iota + d) & (_L - 1) for d in range(_L)]

    def transpose_block(s):
        sbuf, obuf = sbufs[s], obufs[s]

        def cbody(cb, carry):
            cbase = cb * _L
            for rb in range(_D // _L):
                rowv = iota + _L * rb
                for d in range(_L):
                    colv = perms[d] + cbase
                    v = plsc.load_gather(sbuf, [rowv, colv])
                    plsc.store_scatter(obuf, [colv, rowv], v)
            return carry

        lax.fori_loop(0, _RB // _L, cbody, 0)

    in_copy(0, 0).start()

    def step(jj, carry):
        for p in range(2):
            j = jj * 2 + p
            s = p

            @pl.when(j + 1 < _JPW)
            def _():
                in_copy(j + 1, 1 - s).start()

            in_copy(j, s).wait()

            @pl.when(j >= 2)
            def _():
                out_copy(j - 2, s).wait()

            transpose_block(s)
            out_copy(j, s).start()
        return carry

    lax.fori_loop(0, _JPW // 2, step, 0)

    out_copy(_JPW - 2, 0).wait()
    out_copy(_JPW - 1, 1).wait()

    @pl.when(wid == 0)
    def _():
        pltpu.sync_copy(tail_hbm, tbuf)

        def tailrow(j, carry):
            for k in range(_D // _L):
                obuf0[j, pl.ds(_L * k, _L)] = tbuf[j, pl.ds(_L * k, _L)]
            return carry

        lax.fori_loop(0, _D, tailrow, 0)
        pltpu.sync_copy(obuf0.at[pl.ds(0, _D)], tp_hbm.at[pl.ds(_V - _D, _D)])


@functools.partial(jax.jit, donate_argnums=())
def _relayout(wt, tail):
    mesh = plsc.VectorSubcoreMesh(core_axis_name="c", subcore_axis_name="s")
    run = functools.partial(
        pl.kernel,
        mesh=mesh,
        out_type=jax.ShapeDtypeStruct((_V, _DP), jnp.float32),
        compiler_params=pltpu.CompilerParams(needs_layout_passes=False),
        scratch_types=[
            pltpu.VMEM((_D, _RB), jnp.float32),
            pltpu.VMEM((_D, _RB), jnp.float32),
            pltpu.VMEM((_RB, _DP), jnp.float32),
            pltpu.VMEM((_RB, _DP), jnp.float32),
            pltpu.VMEM((_D, _D), jnp.float32),
            pltpu.SemaphoreType.DMA,
            pltpu.SemaphoreType.DMA,
            pltpu.SemaphoreType.DMA,
            pltpu.SemaphoreType.DMA,
        ],
    )(_transpose_body)
    return run(wt, tail)


_BPW = _B // _NW
_CHUNK = 256
_NCHUNK = _BPW // _CHUNK


def _gather_body(idx_hbm, table_hbm, out_hbm, idx_v, rows0, rows1,
                 gsem0, gsem1, wsem0, wsem1):
    wid = lax.axis_index("s") * _NC + lax.axis_index("c")
    base = wid * _BPW
    pltpu.sync_copy(idx_hbm.at[pl.ds(base, _BPW)], idx_v)

    bufs = (rows0, rows1)
    gsems = (gsem0, gsem1)
    wsems = (wsem0, wsem1)

    def gather_copy(c, b):
        return pltpu.make_async_copy(
            table_hbm.at[idx_v.at[pl.ds(c * _CHUNK, _CHUNK)]],
            bufs[b], gsems[b],
        )

    def write_copy(c, b):
        return pltpu.make_async_copy(
            bufs[b], out_hbm.at[pl.ds(base + c * _CHUNK, _CHUNK)], wsems[b],
        )

    gather_copy(0, 0).start()
    gather_copy(1, 1).start()
    gather_copy(0, 0).wait()
    write_copy(0, 0).start()

    def step(g, carry):
        for p in range(2):
            c = 1 + g * 2 + p
            b, ob = (1 + p) % 2, p % 2
            write_copy(c - 1, ob).wait()
            gather_copy(c + 1, ob).start()
            gather_copy(c, b).wait()
            write_copy(c, b).start()
        return carry

    lax.fori_loop(0, (_NCHUNK - 2) // 2, step, 0)

    c = _NCHUNK - 1
    gather_copy(c, c % 2).wait()
    write_copy(c, c % 2).start()
    write_copy(c - 1, (c - 1) % 2).wait()
    write_copy(c, c % 2).wait()


@functools.partial(jax.jit, donate_argnums=())
def _embedding_gather(x_flat, emb_weight):
    mesh = plsc.VectorSubcoreMesh(core_axis_name="c", subcore_axis_name="s")
    table_pad = _relayout(emb_weight.T, emb_weight[_V - _D:, :])
    run = functools.partial(
        pl.kernel,
        mesh=mesh,
        out_type=jax.ShapeDtypeStruct((_B, _DP), jnp.float32),
        scratch_types=[
            pltpu.VMEM((_BPW,), jnp.int32),
            pltpu.VMEM((_CHUNK, _DP), jnp.float32),
            pltpu.VMEM((_CHUNK, _DP), jnp.float32),
            pltpu.SemaphoreType.DMA,
            pltpu.SemaphoreType.DMA,
            pltpu.SemaphoreType.DMA,
            pltpu.SemaphoreType.DMA,
        ],
    )(_gather_body)
    return run(x_flat, table_pad)


def kernel(x, emb_weight):
    out = _embedding_gather(x.reshape(-1).astype(jnp.int32), emb_weight)
    return out[:, :_D].reshape(x.shape + (_D,))


# speedup vs baseline: 2.0261x; 1.3388x over previous
import functools

import jax
import jax.numpy as jnp
from jax import lax
from jax.experimental import pallas as pl
from jax.experimental.pallas import tpu as pltpu
from jax.experimental.pallas import tpu_sc as plsc

_V = 1_000_000
_B = 4096 * 200
_D = 64
_DP = 128
_NC = 2
_NS = 16
_NW = _NC * _NS
_L = 16


_RB = 128
_JPW = 246


def _transpose_body(wt_hbm, tail_hbm, tp_hbm, sbuf0, sbuf1, obuf0, obuf1,
                    tbuf, isem0, isem1, osem0, osem1):
    wid = lax.axis_index("s") * _NC + lax.axis_index("c")
    sbufs, obufs = (sbuf0, sbuf1), (obuf0, obuf1)
    isems, osems = (isem0, isem1), (osem0, osem1)

    _LAST = (_V // _RB - 1) * _RB

    def r0_of(j):
        return jnp.minimum((wid + _NW * j) * _RB, _LAST)

    def in_copy(j, s):
        return pltpu.make_async_copy(
            wt_hbm.at[:, pl.ds(r0_of(j), _RB)], sbufs[s], isems[s])

    def out_copy(j, s):
        return pltpu.make_async_copy(
            obufs[s], tp_hbm.at[pl.ds(r0_of(j), _RB)], osems[s])

    iota = lax.iota(jnp.int32, _L)
    perms = [(iota + d) & (_L - 1) for d in range(_L)]

    def transpose_block(s):
        sbuf, obuf = sbufs[s], obufs[s]

        def cbody(cb, carry):
            cbase = cb * _L
            colvs = [perms[d] + cbase for d in range(_L)]
            for rb in range(_D // _L):
                rowv = iota + _L * rb
                vs = [plsc.load_gather(sbuf, [rowv, colvs[d]])
                      for d in range(_L)]
                for d in range(_L):
                    plsc.store_scatter(obuf, [colvs[d], rowv], vs[d])
            return carry

        lax.fori_loop(0, _RB // _L, cbody, 0)

    in_copy(0, 0).start()

    def step(jj, carry):
        for p in range(2):
            j = jj * 2 + p
            s = p

            @pl.when(j + 1 < _JPW)
            def _():
                in_copy(j + 1, 1 - s).start()

            in_copy(j, s).wait()

            @pl.when(j >= 2)
            def _():
                out_copy(j - 2, s).wait()

            transpose_block(s)
            out_copy(j, s).start()
        return carry

    lax.fori_loop(0, _JPW // 2, step, 0)

    out_copy(_JPW - 2, 0).wait()
    out_copy(_JPW - 1, 1).wait()

    @pl.when(wid == 0)
    def _():
        pltpu.sync_copy(tail_hbm, tbuf)

        def tailrow(j, carry):
            for k in range(_D // _L):
                obuf0[j, pl.ds(_L * k, _L)] = tbuf[j, pl.ds(_L * k, _L)]
            return carry

        lax.fori_loop(0, _D, tailrow, 0)
        pltpu.sync_copy(obuf0.at[pl.ds(0, _D)], tp_hbm.at[pl.ds(_V - _D, _D)])


@functools.partial(jax.jit, donate_argnums=())
def _relayout(wt, tail):
    mesh = plsc.VectorSubcoreMesh(core_axis_name="c", subcore_axis_name="s")
    run = functools.partial(
        pl.kernel,
        mesh=mesh,
        out_type=jax.ShapeDtypeStruct((_V, _DP), jnp.float32),
        compiler_params=pltpu.CompilerParams(needs_layout_passes=False),
        scratch_types=[
            pltpu.VMEM((_D, _RB), jnp.float32),
            pltpu.VMEM((_D, _RB), jnp.float32),
            pltpu.VMEM((_RB, _DP), jnp.float32),
            pltpu.VMEM((_RB, _DP), jnp.float32),
            pltpu.VMEM((_D, _D), jnp.float32),
            pltpu.SemaphoreType.DMA,
            pltpu.SemaphoreType.DMA,
            pltpu.SemaphoreType.DMA,
            pltpu.SemaphoreType.DMA,
        ],
    )(_transpose_body)
    return run(wt, tail)


_BPW = _B // _NW
_CHUNK = 256
_NCHUNK = _BPW // _CHUNK


def _gather_body(idx_hbm, table_hbm, out_hbm, idx_v, rows0, rows1,
                 gsem0, gsem1, wsem0, wsem1):
    wid = lax.axis_index("s") * _NC + lax.axis_index("c")
    base = wid * _BPW
    pltpu.sync_copy(idx_hbm.at[pl.ds(base, _BPW)], idx_v)

    bufs = (rows0, rows1)
    gsems = (gsem0, gsem1)
    wsems = (wsem0, wsem1)

    def gather_copy(c, b):
        return pltpu.make_async_copy(
            table_hbm.at[idx_v.at[pl.ds(c * _CHUNK, _CHUNK)]],
            bufs[b], gsems[b],
        )

    def write_copy(c, b):
        return pltpu.make_async_copy(
            bufs[b], out_hbm.at[pl.ds(base + c * _CHUNK, _CHUNK)], wsems[b],
        )

    gather_copy(0, 0).start()
    gather_copy(1, 1).start()
    gather_copy(0, 0).wait()
    write_copy(0, 0).start()

    def step(g, carry):
        for p in range(2):
            c = 1 + g * 2 + p
            b, ob = (1 + p) % 2, p % 2
            write_copy(c - 1, ob).wait()
            gather_copy(c + 1, ob).start()
            gather_copy(c, b).wait()
            write_copy(c, b).start()
        return carry

    lax.fori_loop(0, (_NCHUNK - 2) // 2, step, 0)

    c = _NCHUNK - 1
    gather_copy(c, c % 2).wait()
    write_copy(c, c % 2).start()
    write_copy(c - 1, (c - 1) % 2).wait()
    write_copy(c, c % 2).wait()


@functools.partial(jax.jit, donate_argnums=())
def _embedding_gather(x_flat, emb_weight):
    mesh = plsc.VectorSubcoreMesh(core_axis_name="c", subcore_axis_name="s")
    table_pad = _relayout(emb_weight.T, emb_weight[_V - _D:, :])
    run = functools.partial(
        pl.kernel,
        mesh=mesh,
        out_type=jax.ShapeDtypeStruct((_B, _DP), jnp.float32),
        scratch_types=[
            pltpu.VMEM((_BPW,), jnp.int32),
            pltpu.VMEM((_CHUNK, _DP), jnp.float32),
            pltpu.VMEM((_CHUNK, _DP), jnp.float32),
            pltpu.SemaphoreType.DMA,
            pltpu.SemaphoreType.DMA,
            pltpu.SemaphoreType.DMA,
            pltpu.SemaphoreType.DMA,
        ],
    )(_gather_body)
    return run(x_flat, table_pad)


def kernel(x, emb_weight):
    out = _embedding_gather(x.reshape(-1).astype(jnp.int32), emb_weight)
    return out[:, :_D].reshape(x.shape + (_D,))


# speedup vs baseline: 2.6725x; 1.3191x over previous
import functools

import jax
import jax.numpy as jnp
from jax import lax
from jax.experimental import pallas as pl
from jax.experimental.pallas import tpu as pltpu
from jax.experimental.pallas import tpu_sc as plsc

_V = 1_000_000
_B = 4096 * 200
_D = 64
_DP = 128
_NC = 2
_NS = 16
_NW = _NC * _NS
_L = 16


_RB = 128
_JPW = 246


def _transpose_body(wt_hbm, tail_hbm, tp_hbm, sbuf0, sbuf1, obuf0, obuf1,
                    tbuf, isem0, isem1, osem0, osem1):
    wid = lax.axis_index("s") * _NC + lax.axis_index("c")
    sbufs, obufs = (sbuf0, sbuf1), (obuf0, obuf1)
    isems, osems = (isem0, isem1), (osem0, osem1)

    _LAST = (_V // _RB - 1) * _RB

    def r0_of(j):
        return jnp.minimum((wid + _NW * j) * _RB, _LAST)

    def in_copy(j, s):
        return pltpu.make_async_copy(
            wt_hbm.at[:, pl.ds(r0_of(j), _RB)], sbufs[s], isems[s])

    def out_copy(j, s):
        return pltpu.make_async_copy(
            obufs[s], tp_hbm.at[pl.ds(r0_of(j), _RB)], osems[s])

    iota = lax.iota(jnp.int32, _L)
    perms = [(iota + d) & (_L - 1) for d in range(_L)]

    def transpose_block(s):
        sbuf, obuf = sbufs[s], obufs[s]

        def cbody(cb, carry):
            cbase = cb * _L
            colvs = [perms[d] + cbase for d in range(_L)]
            for rb in range(_D // _L):
                rowv = iota + _L * rb
                vs = [plsc.load_gather(sbuf, [rowv, colvs[d]])
                      for d in range(_L)]
                for d in range(_L):
                    plsc.store_scatter(obuf, [colvs[d], rowv], vs[d])
            return carry

        lax.fori_loop(0, _RB // _L, cbody, 0)

    in_copy(0, 0).start()

    def step(jj, carry):
        for p in range(2):
            j = jj * 2 + p
            s = p

            @pl.when(j + 1 < _JPW)
            def _():
                in_copy(j + 1, 1 - s).start()

            in_copy(j, s).wait()

            @pl.when(j >= 2)
            def _():
                out_copy(j - 2, s).wait()

            transpose_block(s)
            out_copy(j, s).start()
        return carry

    lax.fori_loop(0, _JPW // 2, step, 0)

    out_copy(_JPW - 2, 0).wait()
    out_copy(_JPW - 1, 1).wait()

    @pl.when(wid == 0)
    def _():
        pltpu.sync_copy(tail_hbm, tbuf)

        def tailrow(j, carry):
            for k in range(_D // _L):
                obuf0[j, pl.ds(_L * k, _L)] = tbuf[j, pl.ds(_L * k, _L)]
            return carry

        lax.fori_loop(0, _D, tailrow, 0)
        pltpu.sync_copy(obuf0.at[pl.ds(0, _D)], tp_hbm.at[pl.ds(_V - _D, _D)])


@functools.partial(jax.jit, donate_argnums=())
def _relayout(wt, tail):
    mesh = plsc.VectorSubcoreMesh(core_axis_name="c", subcore_axis_name="s")
    run = functools.partial(
        pl.kernel,
        mesh=mesh,
        out_type=jax.ShapeDtypeStruct((_V, _DP), jnp.float32),
        compiler_params=pltpu.CompilerParams(needs_layout_passes=False),
        scratch_types=[
            pltpu.VMEM((_D, _RB), jnp.float32),
            pltpu.VMEM((_D, _RB), jnp.float32),
            pltpu.VMEM((_RB, _DP), jnp.float32),
            pltpu.VMEM((_RB, _DP), jnp.float32),
            pltpu.VMEM((_D, _D), jnp.float32),
            pltpu.SemaphoreType.DMA,
            pltpu.SemaphoreType.DMA,
            pltpu.SemaphoreType.DMA,
            pltpu.SemaphoreType.DMA,
        ],
    )(_transpose_body)
    return run(wt, tail)


_T = 200
_NTB = _T // 8


def _gather_body(xt_hbm, table_hbm, outq_hbm, xbuf0, xbuf1, g0, g1, o0, o1,
                 xsem0, xsem1, gsem0, gsem1, wsem0, wsem1):
    wid = lax.axis_index("s") * _NC + lax.axis_index("c")
    col0 = wid * 128
    xbufs, xsems = (xbuf0, xbuf1), (xsem0, xsem1)
    gbufs, gsems = (g0, g1), (gsem0, gsem1)
    obufs, wsems = (o0, o1), (wsem0, wsem1)

    def xcopy(tb, s):
        return pltpu.make_async_copy(
            xt_hbm.at[pl.ds(tb * 8, 8), pl.ds(col0, 128)], xbufs[s], xsems[s])

    def gather(idx_ref, b):
        return pltpu.make_async_copy(table_hbm.at[idx_ref], gbufs[b], gsems[b])

    def write(t, b):
        return pltpu.make_async_copy(
            obufs[b], outq_hbm.at[t, :, pl.ds(col0, 128)], wsems[b])

    iota = lax.iota(jnp.int32, _L)
    perms = [(iota + d) & (_L - 1) for d in range(_L)]

    def transpose_unit(b):
        gbuf, obuf = gbufs[b], obufs[b]

        def cbody(cb, carry):
            rowv = iota + _L * cb
            for rb in range(_D // _L):
                colvs = [perms[d] + _L * rb for d in range(_L)]
                vs = [plsc.load_gather(gbuf, [rowv, colvs[d]])
                      for d in range(_L)]
                for d in range(_L):
                    plsc.store_scatter(obuf, [colvs[d], rowv], vs[d])
            return carry

        lax.fori_loop(0, 128 // _L, cbody, 0)

    def do_block(tb, xb):
        for u in range(8):
            t = tb * 8 + u
            b = u % 2

            if u == 0:
                @pl.when(tb + 1 < _NTB)
                def _():
                    xcopy(tb + 1, 1 - xb).start()
            if u == 7:
                @pl.when(tb + 1 < _NTB)
                def _():
                    xcopy(tb + 1, 1 - xb).wait()

            nidx = xbufs[xb].at[u + 1] if u < 7 else xbufs[1 - xb].at[0]

            @pl.when(t + 1 < _T)
            def _():
                gather(nidx, (u + 1) % 2).start()

            @pl.when(t < _T)
            def _():
                gather(xbufs[xb].at[u], b).wait()

                @pl.when(t >= 2)
                def _():
                    write(t - 2, b).wait()

                transpose_unit(b)
                write(t, b).start()

    xcopy(0, 0).start()
    xcopy(0, 0).wait()
    gather(xbufs[0].at[0], 0).start()

    def step(q, carry):
        do_block(2 * q, 0)
        do_block(2 * q + 1, 1)
        return carry

    lax.fori_loop(0, (_NTB + 2) // 2, step, 0)

    write(_T - 2, 0).wait()
    write(_T - 1, 1).wait()


@functools.partial(jax.jit, donate_argnums=())
def _embedding_gather(xt, emb_weight):
    mesh = plsc.VectorSubcoreMesh(core_axis_name="c", subcore_axis_name="s")
    table_pad = _relayout(emb_weight.T, emb_weight[_V - _D:, :])
    run = functools.partial(
        pl.kernel,
        mesh=mesh,
        out_type=jax.ShapeDtypeStruct((_T, _D, 4096), jnp.float32),
        compiler_params=pltpu.CompilerParams(needs_layout_passes=False),
        scratch_types=[
            pltpu.VMEM((8, 128), jnp.int32),
            pltpu.VMEM((8, 128), jnp.int32),
            pltpu.VMEM((128, 128), jnp.float32),
            pltpu.VMEM((128, 128), jnp.float32),
            pltpu.VMEM((_D, 128), jnp.float32),
            pltpu.VMEM((_D, 128), jnp.float32),
            pltpu.SemaphoreType.DMA,
            pltpu.SemaphoreType.DMA,
            pltpu.SemaphoreType.DMA,
            pltpu.SemaphoreType.DMA,
            pltpu.SemaphoreType.DMA,
            pltpu.SemaphoreType.DMA,
        ],
    )(_gather_body)
    return run(xt, table_pad)


def kernel(x, emb_weight):
    out_q = _embedding_gather(x.T.astype(jnp.int32), emb_weight)
    return jnp.transpose(out_q, (2, 0, 1))
